# baseline (speedup 1.0000x reference)
import functools
import math

import jax
import jax.numpy as jnp
from jax import lax
from jax.experimental import pallas as pl
from jax.experimental.pallas import tpu as pltpu
from jax.experimental.pallas import tpu_sc as plsc

B = 16384
NC, NS = 2, 16
NW = NC * NS
BPW = B // NW
CHUNK = 128
NCH = BPW // CHUNK

D_ITEM, D_CAT = 32, 16
H, OUT = 256, 64
_BN = 1.0 / math.sqrt(1.0 + 1e-5)
_DIMS = (D_ITEM, D_CAT, D_CAT, D_CAT, D_CAT)

_sc_mesh = plsc.VectorSubcoreMesh(
    core_axis_name="c", subcore_axis_name="s", num_cores=NC, num_subcores=NS)


def _sc_gather_body(c0, c1, c2, c3, c4, t0, t1, t2, t3, t4,
                    e0, e1, e2, e3, e4,
                    i0, i1, i2, i3, i4,
                    b0a, b0b, b1a, b1b, b2a, b2b, b3a, b3b, b4a, b4b,
                    s0a, s0b, s1a, s1b, s2a, s2b, s3a, s3b, s4a, s4b):
    wid = lax.axis_index("s") * NC + lax.axis_index("c")
    base = wid * BPW
    cols = (c0, c1, c2, c3, c4)
    idxs = (i0, i1, i2, i3, i4)
    bufs = ((b0a, b0b), (b1a, b1b), (b2a, b2b), (b3a, b3b), (b4a, b4b))
    sems = ((s0a, s0b), (s1a, s1b), (s2a, s2b), (s3a, s3b), (s4a, s4b))
    tabs = (t0, t1, t2, t3, t4)
    outs = (e0, e1, e2, e3, e4)

    for t in range(5):
        for j in range(NCH):
            pltpu.sync_copy(cols[t].at[pl.ds(base + j * CHUNK, CHUNK)],
                            idxs[t].at[j])

    h = [[None, None] for _ in range(5)]

    def drain(j):
        sl = pl.ds(base + j * CHUNK, CHUNK)
        for t in range(5):
            h[t][j % 2].wait()
            pltpu.sync_copy(bufs[t][j % 2], outs[t].at[sl])

    for j in range(NCH):
        for t in range(5):
            h[t][j % 2] = pltpu.async_copy(
                tabs[t].at[idxs[t].at[j]], bufs[t][j % 2], sems[t][j % 2])
        if j > 0:
            drain(j - 1)
    drain(NCH - 1)


_sc_gather = pl.kernel(
    _sc_gather_body,
    out_type=[jax.ShapeDtypeStruct((B, d), jnp.float32) for d in _DIMS],
    mesh=_sc_mesh,
    scratch_types=(
        [pltpu.VMEM((NCH, CHUNK), jnp.int32) for _ in range(5)]
        + [pltpu.VMEM((CHUNK, d), jnp.float32) for d in _DIMS for _ in (0, 1)]
        + [pltpu.SemaphoreType.DMA for _ in range(10)]),
    compiler_params=pltpu.CompilerParams(use_tc_tiling_on_sc=False),
)


def _mlp_body(e0, e1, e2, e3, e4, dnT, w1a, w1b, w1c, w1d, w1e, w1f,
              b1, gm, bt, w2t, b2, out):
    h = jnp.dot(e0[...], w1a[...], preferred_element_type=jnp.float32)
    h = h + jnp.dot(e1[...], w1b[...], preferred_element_type=jnp.float32)
    h = h + jnp.dot(e2[...], w1c[...], preferred_element_type=jnp.float32)
    h = h + jnp.dot(e3[...], w1d[...], preferred_element_type=jnp.float32)
    h = h + jnp.dot(e4[...], w1e[...], preferred_element_type=jnp.float32)
    h = h + lax.dot_general(dnT[...], w1f[...], (((0,), (0,)), ((), ())),
                            preferred_element_type=jnp.float32)
    h = (h + b1[...]) * (_BN * gm[...]) + bt[...]
    h = jnp.maximum(h, 0.0)
    o = lax.dot_general(h, w2t[...], (((1,), (1,)), ((), ())),
                        preferred_element_type=jnp.float32) + b2[...]
    nrm = jnp.sqrt(jnp.sum(o * o, axis=1, keepdims=True))
    out[...] = o / jnp.maximum(nrm, 1e-12)


def _mlp(e0, e1, e2, e3, e4, dnT, w1a, w1b, w1c, w1d, w1e, w1f,
         b1, gm, bt, w2t, b2, block_rows=2048):
    grid = (B // block_rows,)

    def row_spec(d):
        return pl.BlockSpec((block_rows, d), lambda i: (i, 0))

    def full_spec(shape):
        return pl.BlockSpec(shape, lambda i: (0,) * len(shape))

    return pl.pallas_call(
        _mlp_body,
        grid=grid,
        in_specs=[
            row_spec(D_ITEM), row_spec(D_CAT), row_spec(D_CAT),
            row_spec(D_CAT), row_spec(D_CAT),
            pl.BlockSpec((3, block_rows), lambda i: (0, i)),
            full_spec((D_ITEM, H)), full_spec((D_CAT, H)),
            full_spec((D_CAT, H)), full_spec((D_CAT, H)),
            full_spec((D_CAT, H)), full_spec((3, H)),
            full_spec((1, H)), full_spec((1, H)), full_spec((1, H)),
            full_spec((OUT, H)), full_spec((1, OUT)),
        ],
        out_specs=pl.BlockSpec((block_rows, OUT), lambda i: (i, 0)),
        out_shape=jax.ShapeDtypeStruct((B, OUT), jnp.float32),
    )(e0, e1, e2, e3, e4, dnT, w1a, w1b, w1c, w1d, w1e, w1f,
      b1, gm, bt, w2t, b2)


def kernel(item_cat, item_dense, item_emb, cat_l1_emb, cat_l2_emb,
           brand_emb, price_emb, W1, b1, gamma, beta, W2, b2):
    ic = item_cat.astype(jnp.int32)
    c0, c1, c2, c3, c4 = (ic[:, j] for j in range(5))
    price16 = jnp.pad(price_emb, ((0, 0), (0, 8)))

    e0, e1, e2, e3, e4 = _sc_gather(
        c0, c1, c2, c3, c4,
        item_emb, cat_l1_emb, cat_l2_emb, brand_emb, price16)

    w1a = W1[0:32]
    w1b = W1[32:48]
    w1c = W1[48:64]
    w1d = W1[64:80]
    w1e = jnp.pad(W1[80:88], ((0, 8), (0, 0)))
    w1f = W1[88:91]

    return _mlp(e0, e1, e2, e3, e4, item_dense.T,
                w1a, w1b, w1c, w1d, w1e, w1f,
                b1.reshape(1, H), gamma.reshape(1, H), beta.reshape(1, H),
                W2.T, b2.reshape(1, OUT))
